# Initial kernel scaffold; baseline (speedup 1.0000x reference)
import jax
import jax.numpy as jnp
from jax.experimental import pallas as pl


def kernel(cards, card_embed, rank_embed, suit_embed):
    raise NotImplementedError("write your pallas kernel here")


# speedup vs baseline: 8.3910x; 8.3910x over previous
import functools

import jax
import jax.numpy as jnp
from jax import lax
from jax.experimental import pallas as pl
from jax.experimental.pallas import tpu as pltpu
from jax.experimental.pallas import tpu_sc as plsc

EMBED_DIM = 64
BATCH = 16384
HIST = 50
NUM_CARDS = 52
TOTAL = BATCH * HIST
CHUNK = 128


def _fuse_body(card_ref, rank_ref, suit_ref, out_ref):
    ci = lax.broadcasted_iota(jnp.int32, (NUM_CARDS, 13), 0)
    ri = lax.broadcasted_iota(jnp.int32, (NUM_CARDS, 13), 1)
    oh_rank = (ci // 4 == ri).astype(jnp.float32)
    cs = lax.broadcasted_iota(jnp.int32, (NUM_CARDS, 4), 0)
    si = lax.broadcasted_iota(jnp.int32, (NUM_CARDS, 4), 1)
    oh_suit = (cs % 4 == si).astype(jnp.float32)
    out_ref[...] = (
        card_ref[...]
        + lax.dot(oh_rank, rank_ref[...], precision=lax.Precision.HIGHEST)
        + lax.dot(oh_suit, suit_ref[...], precision=lax.Precision.HIGHEST)
    )


def _fuse_tables(card_embed, rank_embed, suit_embed):
    return pl.pallas_call(
        _fuse_body,
        out_shape=jax.ShapeDtypeStruct((NUM_CARDS, EMBED_DIM), jnp.float32),
    )(card_embed, rank_embed, suit_embed)


def _make_gather():
    try:
        info = plsc.get_sparse_core_info()
        nc, ns = info.num_cores, info.num_subcores
    except Exception:
        nc, ns = 2, 16
    nw = nc * ns
    b_per_w = TOTAL // nw
    n_chunks = b_per_w // CHUNK
    assert b_per_w % CHUNK == 0

    mesh = plsc.VectorSubcoreMesh(
        core_axis_name="c", subcore_axis_name="s", num_cores=nc, num_subcores=ns
    )

    @functools.partial(
        pl.kernel,
        mesh=mesh,
        out_type=jax.ShapeDtypeStruct((TOTAL, EMBED_DIM), jnp.float32),
        scratch_types=[
            pltpu.VMEM((n_chunks, CHUNK), jnp.int32),
            pltpu.VMEM((CHUNK, EMBED_DIM), jnp.float32),
            pltpu.SemaphoreType.DMA,
        ],
        compiler_params=pltpu.CompilerParams(use_tc_tiling_on_sc=False),
    )
    def gather(idx_hbm, fused_hbm, out_hbm, idx_v, rows_v, sem):
        wid = lax.axis_index("s") * nc + lax.axis_index("c")
        base = wid * b_per_w
        pltpu.sync_copy(idx_hbm.at[wid], idx_v)

        def body(j, carry):
            pltpu.async_copy(fused_hbm.at[idx_v.at[j]], rows_v, sem).wait()
            pltpu.sync_copy(rows_v, out_hbm.at[pl.ds(base + j * CHUNK, CHUNK)])
            return carry

        lax.fori_loop(0, n_chunks, body, 0)

    return gather, nw, n_chunks


def kernel(cards, card_embed, rank_embed, suit_embed):
    fused = _fuse_tables(card_embed, rank_embed, suit_embed)
    gather, nw, n_chunks = _make_gather()
    idx = cards.astype(jnp.int32).reshape(nw, n_chunks, CHUNK)
    out = gather(idx, fused)
    return out.reshape(BATCH, HIST, EMBED_DIM)


# speedup vs baseline: 8.6143x; 1.0266x over previous
import functools

import jax
import jax.numpy as jnp
from jax import lax
from jax.experimental import pallas as pl
from jax.experimental.pallas import tpu as pltpu
from jax.experimental.pallas import tpu_sc as plsc

EMBED_DIM = 64
BATCH = 16384
HIST = 50
NUM_CARDS = 52
TOTAL = BATCH * HIST
CHUNK = 128
MACRO = 512


def _fuse_body(card_ref, rank_ref, suit_ref, out_ref):
    ci = lax.broadcasted_iota(jnp.int32, (NUM_CARDS, 13), 0)
    ri = lax.broadcasted_iota(jnp.int32, (NUM_CARDS, 13), 1)
    oh_rank = (ci // 4 == ri).astype(jnp.float32)
    cs = lax.broadcasted_iota(jnp.int32, (NUM_CARDS, 4), 0)
    si = lax.broadcasted_iota(jnp.int32, (NUM_CARDS, 4), 1)
    oh_suit = (cs % 4 == si).astype(jnp.float32)
    out_ref[...] = (
        card_ref[...]
        + lax.dot(oh_rank, rank_ref[...], precision=lax.Precision.HIGHEST)
        + lax.dot(oh_suit, suit_ref[...], precision=lax.Precision.HIGHEST)
    )


def _fuse_tables(card_embed, rank_embed, suit_embed):
    return pl.pallas_call(
        _fuse_body,
        out_shape=jax.ShapeDtypeStruct((NUM_CARDS, EMBED_DIM), jnp.float32),
    )(card_embed, rank_embed, suit_embed)


def _make_gather():
    try:
        info = plsc.get_sparse_core_info()
        nc, ns = info.num_cores, info.num_subcores
    except Exception:
        nc, ns = 2, 16
    nw = nc * ns
    b_per_w = TOTAL // nw
    n_chunks = b_per_w // CHUNK
    assert b_per_w % CHUNK == 0

    sub = MACRO // CHUNK
    n_macro = b_per_w // MACRO
    assert b_per_w % MACRO == 0 and n_macro % 2 == 0

    mesh = plsc.VectorSubcoreMesh(
        core_axis_name="c", subcore_axis_name="s", num_cores=nc, num_subcores=ns
    )

    @functools.partial(
        pl.kernel,
        mesh=mesh,
        out_type=jax.ShapeDtypeStruct((TOTAL, EMBED_DIM), jnp.float32),
        scratch_types=[
            pltpu.VMEM((n_chunks, CHUNK), jnp.int32),
            pltpu.VMEM((2, MACRO, EMBED_DIM), jnp.float32),
            [pltpu.SemaphoreType.DMA] * 2,
            [pltpu.SemaphoreType.DMA] * 2,
        ],
        compiler_params=pltpu.CompilerParams(use_tc_tiling_on_sc=False),
    )
    def gather(idx_hbm, fused_hbm, out_hbm, idx_v, rows_v, gsem, ssem):
        wid = lax.axis_index("s") * nc + lax.axis_index("c")
        base = wid * b_per_w
        pltpu.sync_copy(idx_hbm.at[wid], idx_v)

        def issue_gathers(g, b):
            for u in range(sub):
                pltpu.async_copy(
                    fused_hbm.at[idx_v.at[g * sub + u]],
                    rows_v.at[b, pl.ds(u * CHUNK, CHUNK)],
                    gsem[b],
                )

        def wait_gathers(b):
            pltpu.make_async_copy(
                out_hbm.at[pl.ds(base, MACRO)], rows_v.at[b], gsem[b]
            ).wait()

        def scatter(g, b):
            pltpu.async_copy(
                rows_v.at[b], out_hbm.at[pl.ds(base + g * MACRO, MACRO)], ssem[b]
            )

        def wait_scatter(b):
            pltpu.make_async_copy(
                rows_v.at[b], out_hbm.at[pl.ds(base, MACRO)], ssem[b]
            ).wait()

        issue_gathers(0, 0)

        def body(G, carry):
            for b in range(2):
                g = 2 * G + b
                nb = 1 - b

                @pl.when(g <= n_macro - 2)
                def _prefetch():
                    @pl.when(g >= 1)
                    def _drain():
                        wait_scatter(nb)

                    issue_gathers(g + 1, nb)

                wait_gathers(b)
                scatter(g, b)
            return carry

        lax.fori_loop(0, n_macro // 2, body, 0)
        wait_scatter(0)
        wait_scatter(1)

    return gather, nw, n_chunks


def kernel(cards, card_embed, rank_embed, suit_embed):
    fused = _fuse_tables(card_embed, rank_embed, suit_embed)
    gather, nw, n_chunks = _make_gather()
    idx = cards.astype(jnp.int32).reshape(nw, n_chunks, CHUNK)
    out = gather(idx, fused)
    return out.reshape(BATCH, HIST, EMBED_DIM)
